# baseline (device time: 34411 ns/iter reference)
import jax
import jax.numpy as jnp
from jax import lax
from jax.experimental import pallas as pl
from jax.experimental.pallas import tpu as pltpu

N_DEV = 8


def kernel(x, Win0, Wout0, Win1, Wout1, Win2, Wout2):
    M, D = x.shape
    CH = M // N_DEV
    bf16 = jnp.bfloat16

    def body(x_ref, win0, wout0, win1, wout1, win2, wout2, out_ref,
             xbuf, partial, rsbuf, rs_send, rs_recv, ag_send, ag_recv):
        me = lax.axis_index("i")

        barrier = pltpu.get_barrier_semaphore()
        for p in range(N_DEV):
            @pl.when(me != p)
            def _(p=p):
                pl.semaphore_signal(
                    barrier, inc=1,
                    device_id=(p,), device_id_type=pl.DeviceIdType.MESH,
                )
        pl.semaphore_wait(barrier, N_DEV - 1)

        def rs_copy(p):
            return pltpu.make_async_remote_copy(
                src_ref=partial.at[pl.ds(p * CH, CH), :],
                dst_ref=rsbuf.at[pl.ds(me * CH, CH), :],
                send_sem=rs_send.at[p],
                recv_sem=rs_recv.at[me],
                device_id=(p,),
                device_id_type=pl.DeviceIdType.MESH,
            )

        def rs_wait_from(s):
            return pltpu.make_async_remote_copy(
                src_ref=partial.at[pl.ds(0, CH), :],
                dst_ref=rsbuf.at[pl.ds(s * CH, CH), :],
                send_sem=rs_send.at[s],
                recv_sem=rs_recv.at[s],
                device_id=(s,),
                device_id_type=pl.DeviceIdType.MESH,
            )

        def ag_copy(p):
            return pltpu.make_async_remote_copy(
                src_ref=xbuf.at[pl.ds(me * CH, CH), :],
                dst_ref=xbuf.at[pl.ds(me * CH, CH), :],
                send_sem=ag_send.at[p],
                recv_sem=ag_recv.at[me],
                device_id=(p,),
                device_id_type=pl.DeviceIdType.MESH,
            )

        def ag_wait_from(s):
            return pltpu.make_async_remote_copy(
                src_ref=xbuf.at[pl.ds(0, CH), :],
                dst_ref=xbuf.at[pl.ds(s * CH, CH), :],
                send_sem=ag_send.at[s],
                recv_sem=ag_recv.at[s],
                device_id=(s,),
                device_id_type=pl.DeviceIdType.MESH,
            )

        wins = [win0, win1, win2]
        wouts = [wout0, wout1, wout2]

        for l in range(3):
            with jax.named_scope(f"compute#l={l}"):
                xv = x_ref[...].astype(bf16) if l == 0 else xbuf[...]
                h = jnp.dot(xv, wins[l][...].astype(bf16),
                            preferred_element_type=jnp.float32)
                h = jnp.maximum(h, 0.0).astype(bf16)
                pv = jnp.dot(h, wouts[l][...].astype(bf16),
                             preferred_element_type=jnp.float32)
                partial[...] = pv.astype(bf16)

            with jax.named_scope(f"rs_send#l={l}"):
                for p in range(N_DEV):
                    @pl.when(me != p)
                    def _(p=p):
                        rs_copy(p).start()
                for p in range(N_DEV):
                    @pl.when(me != p)
                    def _(p=p):
                        rs_copy(p).wait_send()

            with jax.named_scope(f"rs_wait#l={l}"):
                for s in range(N_DEV):
                    @pl.when(me != s)
                    def _(s=s):
                        rs_wait_from(s).wait_recv()

            with jax.named_scope(f"reduce#l={l}"):
                acc = partial[pl.ds(me * CH, CH), :].astype(jnp.float32)
                for s in range(N_DEV):
                    contrib = rsbuf[pl.ds(s * CH, CH), :].astype(jnp.float32)
                    acc = acc + jnp.where(s == me, 0.0, contrib)
                xbuf[pl.ds(me * CH, CH), :] = acc.astype(bf16)

            with jax.named_scope(f"ag_send#l={l}"):
                for p in range(N_DEV):
                    @pl.when(me != p)
                    def _(p=p):
                        ag_copy(p).start()
                for p in range(N_DEV):
                    @pl.when(me != p)
                    def _(p=p):
                        ag_copy(p).wait_send()

            with jax.named_scope(f"ag_wait#l={l}"):
                for s in range(N_DEV):
                    @pl.when(me != s)
                    def _(s=s):
                        ag_wait_from(s).wait_recv()

        out_ref[...] = xbuf[...].astype(jnp.float32)

    return pl.pallas_call(
        body,
        out_shape=jax.ShapeDtypeStruct((M, D), jnp.float32),
        in_specs=[pl.BlockSpec(memory_space=pltpu.VMEM)] * 7,
        out_specs=pl.BlockSpec(memory_space=pltpu.VMEM),
        scratch_shapes=[
            pltpu.VMEM((M, D), bf16),
            pltpu.VMEM((M, D), bf16),
            pltpu.VMEM((M, D), bf16),
            pltpu.SemaphoreType.DMA((N_DEV,)),
            pltpu.SemaphoreType.DMA((N_DEV,)),
            pltpu.SemaphoreType.DMA((N_DEV,)),
            pltpu.SemaphoreType.DMA((N_DEV,)),
        ],
        compiler_params=pltpu.CompilerParams(collective_id=0),
    )(x, Win0, Wout0, Win1, Wout1, Win2, Wout2)
